# Initial kernel scaffold; baseline (speedup 1.0000x reference)
import functools

import jax
import jax.numpy as jnp
from jax import lax
from jax.experimental import pallas as pl
from jax.experimental.pallas import tpu as pltpu
from jax.experimental.pallas import tpu_sc as plsc

NUM_USERS = 100000
NUM_ITEMS = 100000
D = 32
B = 16384
HIST = 50
NG = 20
GEN = 64
DENSE = 8
PAD_IDX = NUM_ITEMS
USER_PAD_IDX = NUM_USERS
NROWS = NUM_USERS + 1


ROWS_BLK = 8192


def _proj_body(ut, ug, it, mg, gn, gw, w_ue, w_ug, w_ie, w_g, w_gen, w_pc,
               w_pd, ucomb, uprojd, icomb, iprojc):
    f32 = jnp.float32

    def dot(a, b):
        return lax.dot_general(a, b, (((1,), (0,)), ((), ())),
                               preferred_element_type=f32)

    gv = lax.dot_general(gw[...], w_g[...], (((0,), (0,)), ((), ())),
                         preferred_element_type=f32)
    ucomb[...] = dot(ut[...], w_ue[...]) + dot(ug[...], w_ug[...])
    uprojd[...] = dot(ut[...], w_pd[...])
    icomb[...] = (dot(it[...], w_ie[...]) + dot(mg[...], gv)
                  + dot(gn[...], w_gen[...]))
    iprojc[...] = dot(it[...], w_pc[...])


def _run_projections(user_table, user_genome, item_table, movie_genres,
                     genome, genre_W, w_ue, w_ug, w_ie, w_g, w_gen, w_pc,
                     w_pd):
    grid = (pl.cdiv(NROWS, ROWS_BLK),)
    row_spec = lambda k: pl.BlockSpec((ROWS_BLK, k), lambda i: (i, 0))
    full_spec = lambda a, b: pl.BlockSpec((a, b), lambda i: (0, 0))
    out_spec = pl.BlockSpec((ROWS_BLK, 1), lambda i: (i, 0))
    out_sd = jax.ShapeDtypeStruct((NROWS, 1), jnp.float32)
    return pl.pallas_call(
        _proj_body,
        grid=grid,
        in_specs=[
            row_spec(D), row_spec(GEN), row_spec(D), row_spec(NG),
            row_spec(GEN), full_spec(D, NG), full_spec(D, 1),
            full_spec(GEN, 1), full_spec(D, 1), full_spec(D, 1),
            full_spec(GEN, 1), full_spec(D, 1), full_spec(D, 1),
        ],
        out_specs=[out_spec] * 4,
        out_shape=[out_sd] * 4,
    )(user_table, user_genome, item_table, movie_genres, genome, genre_W,
      w_ue, w_ug, w_ie, w_g, w_gen, w_pc, w_pd)


NC = 2
NS = 16
L = 16
NW = NC * NS
BPW = B // NW
CHUNK = 128
NCH = BPW // CHUNK


def _sc_body(uids_hbm, mids_hbm, dense_hbm, uhist_hbm, uhrat_hbm, ihist_hbm,
             ihrat_hbm, ucomb_hbm, icomb_hbm, iprojc_hbm, uprojd_hbm,
             wmisc_hbm, out_hbm, uid_v, mid_v, hist_v, rat_v, proj_v,
             ucomb_v, icomb_v, dense_v, out_v, wmisc_v, sem1, sem2, sem3,
             sem4):
    wid = lax.axis_index("s") * NC + lax.axis_index("c")
    base = wid * BPW
    pltpu.sync_copy(uids_hbm.at[pl.ds(base, BPW)], uid_v)
    pltpu.sync_copy(mids_hbm.at[pl.ds(base, BPW)], mid_v)
    pltpu.sync_copy(dense_hbm.at[pl.ds(base, BPW)], dense_v)
    pltpu.sync_copy(wmisc_hbm, wmisc_v)
    cp_uc = pltpu.async_copy(ucomb_hbm.at[uid_v], ucomb_v, sem1)
    cp_ic = pltpu.async_copy(icomb_hbm.at[mid_v], icomb_v, sem2)

    iota = lax.iota(jnp.int32, L)
    zero = jnp.zeros((L,), jnp.float32)

    def do_side(idx_v, hist_hbm, rat_hbm, proj_hbm, pad_val, w_rat, first):
        for ch in range(NCH):
            idx_slice = idx_v.at[pl.ds(ch * CHUNK, CHUNK)]
            g1 = pltpu.async_copy(hist_hbm.at[idx_slice], hist_v, sem3)
            g2 = pltpu.async_copy(rat_hbm.at[idx_slice], rat_v, sem4)
            g1.wait()
            g2.wait()
            g3 = pltpu.async_copy(proj_hbm.at[hist_v], proj_v, sem3)
            g3.wait()
            for bg in range(CHUNK // L):
                rows = jnp.full((L,), bg * L, jnp.int32) + iota

                def hbody(h, carry):
                    cnt, rsum, psum = carry
                    cols = jnp.full((L,), h, jnp.int32)
                    hv = plsc.load_gather(hist_v, [rows, cols])
                    valid = hv != pad_val
                    cnt = cnt + jnp.where(valid, 1.0, 0.0)
                    rv = plsc.load_gather(rat_v, [rows, cols])
                    rsum = rsum + jnp.where(valid, rv, 0.0)
                    psum = psum + plsc.load_gather(proj_v, [rows, cols])
                    return cnt, rsum, psum

                cnt, rsum, psum = lax.fori_loop(0, HIST, hbody,
                                                (zero, zero, zero))
                cnt = jnp.maximum(cnt, 1.0)
                contrib = (psum + w_rat * rsum) / cnt
                off = ch * CHUNK + bg * L
                if first:
                    out_v[pl.ds(off, L)] = contrib
                else:
                    out_v[pl.ds(off, L)] = out_v[pl.ds(off, L)] + contrib

    do_side(uid_v, uhist_hbm, uhrat_hbm, iprojc_hbm, PAD_IDX, wmisc_v[0],
            True)
    do_side(mid_v, ihist_hbm, ihrat_hbm, uprojd_hbm, USER_PAD_IDX,
            wmisc_v[1], False)

    cp_uc.wait()
    cp_ic.wait()
    for bg in range(BPW // L):
        rows = jnp.full((L,), bg * L, jnp.int32) + iota
        acc = (out_v[pl.ds(bg * L, L)] + ucomb_v[pl.ds(bg * L, L)]
               + icomb_v[pl.ds(bg * L, L)] + wmisc_v[2])

        def dbody(k, a):
            cols = jnp.full((L,), k, jnp.int32)
            return a + plsc.load_gather(dense_v, [rows, cols]) * wmisc_v[3 + k]

        acc = lax.fori_loop(0, DENSE, dbody, acc)
        out_v[pl.ds(bg * L, L)] = acc
    pltpu.sync_copy(out_v, out_hbm.at[pl.ds(base, BPW)])


def _sc_run(uids, mids, dense, uhist, uhrat, ihist, ihrat, ucomb, icomb,
            iprojc, uprojd, wmisc):
    mesh = plsc.VectorSubcoreMesh(core_axis_name="c", subcore_axis_name="s",
                                  num_cores=NC, num_subcores=NS)
    f = pl.kernel(
        _sc_body,
        out_type=jax.ShapeDtypeStruct((B,), jnp.float32),
        mesh=mesh,
        scratch_types=[
            pltpu.VMEM((BPW,), jnp.int32),
            pltpu.VMEM((BPW,), jnp.int32),
            pltpu.VMEM((CHUNK, HIST), jnp.int32),
            pltpu.VMEM((CHUNK, HIST), jnp.float32),
            pltpu.VMEM((CHUNK, HIST), jnp.float32),
            pltpu.VMEM((BPW,), jnp.float32),
            pltpu.VMEM((BPW,), jnp.float32),
            pltpu.VMEM((BPW, DENSE), jnp.float32),
            pltpu.VMEM((BPW,), jnp.float32),
            pltpu.VMEM((L,), jnp.float32),
            pltpu.SemaphoreType.DMA,
            pltpu.SemaphoreType.DMA,
            pltpu.SemaphoreType.DMA,
            pltpu.SemaphoreType.DMA,
        ],
    )
    return f(uids, mids, dense, uhist, uhrat, ihist, ihrat, ucomb, icomb,
             iprojc, uprojd, wmisc)


def kernel(uids, mids, dense, user_table, item_table, genre_W, head_W,
           head_b, user_hist, user_hist_rat, item_hist, item_hist_rat,
           movie_genres, genome, user_genome):
    i32 = jnp.int32
    uids = uids.astype(i32)
    mids = mids.astype(i32)
    user_hist = user_hist.astype(i32)
    item_hist = item_hist.astype(i32)

    w = head_W[0]
    col = lambda a, b: w[a:b].reshape(-1, 1)
    w_ue = col(0, 32)
    w_ie = col(32, 64)
    w_pc = col(64, 96)
    w_u_rat = w[96]
    w_pd = col(97, 129)
    w_i_rat = w[129]
    w_g = col(130, 162)
    w_dense = w[162:170]
    w_gen = col(170, 234)
    w_ug = col(234, 298)

    ucomb, uprojd, icomb, iprojc = _run_projections(
        user_table, user_genome, item_table, movie_genres, genome, genre_W,
        w_ue, w_ug, w_ie, w_g, w_gen, w_pc, w_pd)

    wmisc = jnp.concatenate([
        jnp.stack([w_u_rat, w_i_rat, head_b[0]]),
        w_dense,
        jnp.zeros((5,), jnp.float32),
    ])

    return _sc_run(uids, mids, dense, user_hist, user_hist_rat, item_hist,
                   item_hist_rat, ucomb[:, 0], icomb[:, 0], iprojc[:, 0],
                   uprojd[:, 0], wmisc)


# speedup vs baseline: 7.9399x; 7.9399x over previous
import functools

import jax
import jax.numpy as jnp
from jax import lax
from jax.experimental import pallas as pl
from jax.experimental.pallas import tpu as pltpu
from jax.experimental.pallas import tpu_sc as plsc

NUM_USERS = 100000
NUM_ITEMS = 100000
D = 32
B = 16384
HIST = 50
NG = 20
GEN = 64
DENSE = 8
PAD_IDX = NUM_ITEMS
USER_PAD_IDX = NUM_USERS
NROWS = NUM_USERS + 1


ROWS_BLK = 2048


def _proj_body(ut, ug, it, mg, gn, gw, w_ue, w_ug, w_ie, w_g, w_gen, w_pc,
               w_pd, ucomb, uprojd, icomb, iprojc):
    f32 = jnp.float32

    def dot(a, b):
        return lax.dot_general(a, b, (((1,), (0,)), ((), ())),
                               preferred_element_type=f32)

    gv = lax.dot_general(gw[...], w_g[...], (((0,), (0,)), ((), ())),
                         preferred_element_type=f32)
    ucomb[...] = dot(ut[...], w_ue[...]) + dot(ug[...], w_ug[...])
    uprojd[...] = dot(ut[...], w_pd[...])
    icomb[...] = (dot(it[...], w_ie[...]) + dot(mg[...], gv)
                  + dot(gn[...], w_gen[...]))
    iprojc[...] = dot(it[...], w_pc[...])


def _run_projections(user_table, user_genome, item_table, movie_genres,
                     genome, genre_W, w_ue, w_ug, w_ie, w_g, w_gen, w_pc,
                     w_pd):
    grid = (pl.cdiv(NROWS, ROWS_BLK),)
    row_spec = lambda k: pl.BlockSpec((ROWS_BLK, k), lambda i: (i, 0))
    full_spec = lambda a, b: pl.BlockSpec((a, b), lambda i: (0, 0))
    out_spec = pl.BlockSpec((ROWS_BLK, 1), lambda i: (i, 0))
    out_sd = jax.ShapeDtypeStruct((NROWS, 1), jnp.float32)
    return pl.pallas_call(
        _proj_body,
        grid=grid,
        in_specs=[
            row_spec(D), row_spec(GEN), row_spec(D), row_spec(NG),
            row_spec(GEN), full_spec(D, NG), full_spec(D, 1),
            full_spec(GEN, 1), full_spec(D, 1), full_spec(D, 1),
            full_spec(GEN, 1), full_spec(D, 1), full_spec(D, 1),
        ],
        out_specs=[out_spec] * 4,
        out_shape=[out_sd] * 4,
    )(user_table, user_genome, item_table, movie_genres, genome, genre_W,
      w_ue, w_ug, w_ie, w_g, w_gen, w_pc, w_pd)


NC = 2
NS = 16
L = 16
NW = NC * NS
BPW = B // NW
CHUNK = 128
NCH = BPW // CHUNK


def _sc_body(uids_hbm, mids_hbm, dense_hbm, uhist_hbm, uhrat_hbm, ihist_hbm,
             ihrat_hbm, ucomb_hbm, icomb_hbm, iprojc_hbm, uprojd_hbm,
             wmisc_hbm, out_hbm, uid_v, mid_v, fidx_v, histflat_v, ratflat_v,
             projflat_v, div_v, mod_v, ucomb_v, icomb_v, dense_v, out_v,
             wmisc_v, sem1, sem2, sem3, sem4):
    wid = lax.axis_index("s") * NC + lax.axis_index("c")
    base = wid * BPW
    pltpu.sync_copy(uids_hbm.at[pl.ds(base, BPW)], uid_v)
    pltpu.sync_copy(mids_hbm.at[pl.ds(base, BPW)], mid_v)
    pltpu.sync_copy(dense_hbm.at[pl.ds(base, BPW)], dense_v)
    pltpu.sync_copy(wmisc_hbm, wmisc_v)
    cp_uc = pltpu.async_copy(ucomb_hbm.at[uid_v], ucomb_v, sem1)
    cp_ic = pltpu.async_copy(icomb_hbm.at[mid_v], icomb_v, sem2)

    iota = lax.iota(jnp.int32, L)
    zero = jnp.zeros((L,), jnp.float32)
    wm = wmisc_v[...]
    NFLAT = CHUNK * HIST // L

    def dm_body(g, _):
        val = jnp.full((L,), g * L, jnp.int32) + iota
        div_v[pl.ds(g * L, L)] = val // HIST
        mod_v[pl.ds(g * L, L)] = val % HIST
        return _

    lax.fori_loop(0, NFLAT, dm_body, None)

    def do_side(idx_v, histflat_hbm, ratflat_hbm, proj_hbm, pad_val, w_rat,
                first):
        for ch in range(NCH):
            def fidx_body(g, _):
                sl = pl.ds(g * L, L)
                rows = div_v[sl] + (ch * CHUNK)
                uv = plsc.load_gather(idx_v, [rows])
                fidx_v[sl] = uv * HIST + mod_v[sl]
                return _

            lax.fori_loop(0, NFLAT, fidx_body, None)
            g1 = pltpu.async_copy(histflat_hbm.at[fidx_v], histflat_v, sem3)
            g2 = pltpu.async_copy(ratflat_hbm.at[fidx_v], ratflat_v, sem4)
            g1.wait()
            g2.wait()
            g3 = pltpu.async_copy(proj_hbm.at[histflat_v], projflat_v, sem3)
            g3.wait()
            for bg in range(CHUNK // L):
                base50 = (jnp.full((L,), bg * L, jnp.int32) + iota) * HIST

                def hbody(h, carry):
                    cnt, rsum, psum = carry
                    fidx = base50 + h
                    hv = plsc.load_gather(histflat_v, [fidx])
                    valid = hv != pad_val
                    cnt = cnt + jnp.where(valid, 1.0, 0.0)
                    rv = plsc.load_gather(ratflat_v, [fidx])
                    rsum = rsum + jnp.where(valid, rv, 0.0)
                    psum = psum + plsc.load_gather(projflat_v, [fidx])
                    return cnt, rsum, psum

                cnt, rsum, psum = lax.fori_loop(0, HIST, hbody,
                                                (zero, zero, zero))
                cnt = jnp.maximum(cnt, 1.0)
                contrib = (psum + w_rat * rsum) / cnt
                off = ch * CHUNK + bg * L
                if first:
                    out_v[pl.ds(off, L)] = contrib
                else:
                    out_v[pl.ds(off, L)] = out_v[pl.ds(off, L)] + contrib

    do_side(uid_v, uhist_hbm, uhrat_hbm, iprojc_hbm, PAD_IDX, wm[0], True)
    do_side(mid_v, ihist_hbm, ihrat_hbm, uprojd_hbm, USER_PAD_IDX, wm[1],
            False)

    cp_uc.wait()
    cp_ic.wait()
    for bg in range(BPW // L):
        rows = jnp.full((L,), bg * L, jnp.int32) + iota
        acc = (out_v[pl.ds(bg * L, L)] + ucomb_v[pl.ds(bg * L, L)]
               + icomb_v[pl.ds(bg * L, L)] + wm[2])
        for k in range(DENSE):
            cols = jnp.full((L,), k, jnp.int32)
            acc = acc + plsc.load_gather(dense_v, [rows, cols]) * wm[3 + k]
        out_v[pl.ds(bg * L, L)] = acc
    pltpu.sync_copy(out_v, out_hbm.at[pl.ds(base, BPW)])


def _sc_run(uids, mids, dense, uhist, uhrat, ihist, ihrat, ucomb, icomb,
            iprojc, uprojd, wmisc):
    mesh = plsc.VectorSubcoreMesh(core_axis_name="c", subcore_axis_name="s",
                                  num_cores=NC, num_subcores=NS)
    f = pl.kernel(
        _sc_body,
        out_type=jax.ShapeDtypeStruct((B,), jnp.float32),
        mesh=mesh,
        compiler_params=pltpu.CompilerParams(
            needs_layout_passes=False,
            use_tc_tiling_on_sc=False,
        ),
        scratch_types=[
            pltpu.VMEM((BPW,), jnp.int32),
            pltpu.VMEM((BPW,), jnp.int32),
            pltpu.VMEM((CHUNK * HIST,), jnp.int32),
            pltpu.VMEM((CHUNK * HIST,), jnp.int32),
            pltpu.VMEM((CHUNK * HIST,), jnp.float32),
            pltpu.VMEM((CHUNK * HIST,), jnp.float32),
            pltpu.VMEM((CHUNK * HIST,), jnp.int32),
            pltpu.VMEM((CHUNK * HIST,), jnp.int32),
            pltpu.VMEM((BPW,), jnp.float32),
            pltpu.VMEM((BPW,), jnp.float32),
            pltpu.VMEM((BPW, DENSE), jnp.float32),
            pltpu.VMEM((BPW,), jnp.float32),
            pltpu.VMEM((L,), jnp.float32),
            pltpu.SemaphoreType.DMA,
            pltpu.SemaphoreType.DMA,
            pltpu.SemaphoreType.DMA,
            pltpu.SemaphoreType.DMA,
        ],
    )
    return f(uids, mids, dense, uhist, uhrat, ihist, ihrat, ucomb, icomb,
             iprojc, uprojd, wmisc)


def kernel(uids, mids, dense, user_table, item_table, genre_W, head_W,
           head_b, user_hist, user_hist_rat, item_hist, item_hist_rat,
           movie_genres, genome, user_genome):
    i32 = jnp.int32
    uids = uids.astype(i32)
    mids = mids.astype(i32)
    user_hist = user_hist.astype(i32)
    item_hist = item_hist.astype(i32)

    w = head_W[0]
    col = lambda a, b: w[a:b].reshape(-1, 1)
    w_ue = col(0, 32)
    w_ie = col(32, 64)
    w_pc = col(64, 96)
    w_u_rat = w[96]
    w_pd = col(97, 129)
    w_i_rat = w[129]
    w_g = col(130, 162)
    w_dense = w[162:170]
    w_gen = col(170, 234)
    w_ug = col(234, 298)

    ucomb, uprojd, icomb, iprojc = _run_projections(
        user_table, user_genome, item_table, movie_genres, genome, genre_W,
        w_ue, w_ug, w_ie, w_g, w_gen, w_pc, w_pd)

    wmisc = jnp.concatenate([
        jnp.stack([w_u_rat, w_i_rat, head_b[0]]),
        w_dense,
        jnp.zeros((5,), jnp.float32),
    ])

    return _sc_run(uids, mids, dense, user_hist.reshape(-1),
                   user_hist_rat.reshape(-1), item_hist.reshape(-1),
                   item_hist_rat.reshape(-1), ucomb[:, 0], icomb[:, 0],
                   iprojc[:, 0], uprojd[:, 0], wmisc)
